# Initial kernel scaffold; baseline (speedup 1.0000x reference)
import jax
import jax.numpy as jnp
from jax.experimental import pallas as pl


def kernel(x):
    raise NotImplementedError("write your pallas kernel here")


# speedup vs baseline: 1.3988x; 1.3988x over previous
import functools

import jax
import jax.numpy as jnp
import numpy as np
from jax import lax
from jax.experimental import pallas as pl
from jax.experimental.pallas import tpu as pltpu
from jax.experimental.pallas import tpu_sc as plsc

H, W, C = 512, 512, 224
WC = W * C
N_SHIFT = 10

_rng = np.random.default_rng(0)
_allc = _rng.permutation(C)[:N_SHIFT]
_FWD = np.sort(_allc[: N_SHIFT // 2])
_BWD = np.sort(_allc[N_SHIFT // 2:])
_SPECIAL = np.concatenate([_FWD, _BWD])

NG = 8
NS_STRIP = 4
ROWS = H // NG
PX = W // NS_STRIP
CHUNK = PX * C
NSPEC = N_SHIFT * PX
NV = NSPEC // 16
NVH = NV // 2

_IDX = np.concatenate([
    (np.arange(PX)[None, :] * C + _SPECIAL[:, None]).reshape(-1),
    np.pad(_BWD, (0, 11)),
    np.pad(_FWD, (0, 11)),
]).astype(np.int32)

_mesh = plsc.VectorSubcoreMesh(
    core_axis_name="c", subcore_axis_name="s", num_cores=2, num_subcores=16
)


@functools.partial(
    pl.kernel,
    out_type=jax.ShapeDtypeStruct((H, WC), jnp.float32),
    mesh=_mesh,
    compiler_params=pltpu.CompilerParams(
        use_tc_tiling_on_sc=False, needs_layout_passes=False
    ),
    scratch_types=[
        pltpu.VMEM((4 * CHUNK,), jnp.float32),
        pltpu.VMEM((4 * NSPEC,), jnp.float32),
        pltpu.VMEM((NSPEC + 32,), jnp.int32),
        pltpu.VMEM((C,), jnp.float32),
    ],
)
def _sc_shift(x_hbm, idx_hbm, out_hbm, buf, sav, idxv, edge):
    wid = lax.axis_index("s") * 2 + lax.axis_index("c")
    g = wid // NS_STRIP
    st = wid % NS_STRIP
    r0 = g * ROWS
    c0 = st * CHUNK
    lanes = lax.iota(jnp.int32, 16)

    pltpu.sync_copy(idx_hbm, idxv)

    def stream_in(r):
        slot = lax.rem(r, 4)
        pltpu.sync_copy(
            x_hbm.at[r, pl.ds(c0, CHUNK)], buf.at[pl.ds(slot * CHUNK, CHUNK)]
        )

    def extract(r):
        slot = lax.rem(r, 4)

        def body(v, _):
            iv = idxv[pl.ds(v * 16, 16)]
            vals = plsc.load_gather(buf, [slot * CHUNK + iv])
            sav[pl.ds(slot * NSPEC + v * 16, 16)] = vals
            return 0

        lax.fori_loop(0, NV, body, 0)

    def fixup(r):
        slot = lax.rem(r, 4)
        sp1 = lax.rem(r + 1, 4)
        sm1 = lax.rem(r + 3, 4)

        def body(v, _):
            iv = idxv[pl.ds(v * 16, 16)]
            src_slot = jnp.where(v < NVH, sp1, sm1)
            vals = sav[pl.ds(src_slot * NSPEC + v * 16, 16)]
            plsc.store_scatter(buf, [slot * CHUNK + iv], vals)
            return 0

        lax.fori_loop(0, NV, body, 0)

    def stream_out(r):
        slot = lax.rem(r, 4)
        pltpu.sync_copy(
            buf.at[pl.ds(slot * CHUNK, CHUNK)], out_hbm.at[r, pl.ds(c0, CHUNK)]
        )

    def synth_m1():
        @pl.when(st > 0)
        def _():
            pltpu.sync_copy(x_hbm.at[0, pl.ds(c0 - C, C)], edge)

        def body(v, _):
            iv = lanes + (NSPEC // 2 + v * 16 - 1)
            vals = plsc.load_gather(sav, [iv])
            sav[pl.ds(3 * NSPEC + NSPEC // 2 + v * 16, 16)] = vals
            return 0

        lax.fori_loop(0, NVH, body, 0)
        bwd_ch = idxv[pl.ds(NSPEC, 16)]
        vals = plsc.load_gather(edge, [bwd_ch])
        pred = jnp.full((16,), st, jnp.int32) > 0
        vals = jnp.where(pred, vals, jnp.zeros_like(vals))
        plsc.store_scatter(
            sav, [3 * NSPEC + NSPEC // 2 + lanes * PX], vals, mask=lanes < 5
        )

    def synth_p1():
        @pl.when(st < NS_STRIP - 1)
        def _():
            pltpu.sync_copy(x_hbm.at[H - 1, pl.ds(c0 + CHUNK, C)], edge)

        def body(v, _):
            iv = lanes + (3 * NSPEC + v * 16 + 1)
            vals = plsc.load_gather(sav, [iv])
            sav[pl.ds(v * 16, 16)] = vals
            return 0

        lax.fori_loop(0, NVH, body, 0)
        fwd_ch = idxv[pl.ds(NSPEC + 16, 16)]
        vals = plsc.load_gather(edge, [fwd_ch])
        pred = jnp.full((16,), st, jnp.int32) < NS_STRIP - 1
        vals = jnp.where(pred, vals, jnp.zeros_like(vals))
        plsc.store_scatter(sav, [lanes * PX + (PX - 1)], vals, mask=lanes < 5)

    @pl.when(g > 0)
    def _():
        stream_in(r0 - 1)
        extract(r0 - 1)

    stream_in(r0)
    extract(r0)

    @pl.when(g == 0)
    def _():
        synth_m1()

    def row_body(r, _):
        @pl.when(r + 1 < H)
        def _():
            stream_in(r + 1)
            extract(r + 1)

        @pl.when(r + 1 == H)
        def _():
            synth_p1()

        fixup(r)
        stream_out(r)
        return 0

    lax.fori_loop(r0, r0 + ROWS, row_body, 0)


def kernel(x):
    x2d = x.reshape(H, WC)
    out2d = _sc_shift(x2d, jnp.asarray(_IDX))
    return out2d.reshape(H, W, C)


# speedup vs baseline: 1.5421x; 1.1024x over previous
import functools

import jax
import jax.numpy as jnp
import numpy as np
from jax import lax
from jax.experimental import pallas as pl
from jax.experimental.pallas import tpu as pltpu
from jax.experimental.pallas import tpu_sc as plsc

H, W, C = 512, 512, 224
WC = W * C
N_SHIFT = 10

_rng = np.random.default_rng(0)
_allc = _rng.permutation(C)[:N_SHIFT]
_FWD = np.sort(_allc[: N_SHIFT // 2])
_BWD = np.sort(_allc[N_SHIFT // 2:])
_SPECIAL = np.concatenate([_FWD, _BWD])

NG = 8
NS_STRIP = 4
ROWS = H // NG
PX = W // NS_STRIP
CHUNK = PX * C
NSPEC = N_SHIFT * PX
NV = NSPEC // 16
NVH = NV // 2

_IDX = np.concatenate([
    (np.arange(PX)[None, :] * C + _SPECIAL[:, None]).reshape(-1),
    np.pad(_BWD, (0, 11)),
    np.pad(_FWD, (0, 11)),
]).astype(np.int32)

_mesh = plsc.VectorSubcoreMesh(
    core_axis_name="c", subcore_axis_name="s", num_cores=2, num_subcores=16
)


@functools.partial(
    pl.kernel,
    out_type=jax.ShapeDtypeStruct((H, WC), jnp.float32),
    mesh=_mesh,
    compiler_params=pltpu.CompilerParams(
        use_tc_tiling_on_sc=False, needs_layout_passes=False
    ),
    scratch_types=[
        pltpu.VMEM((4 * CHUNK,), jnp.float32),
        pltpu.VMEM((4 * NSPEC,), jnp.float32),
        pltpu.VMEM((NSPEC + 32,), jnp.int32),
        pltpu.VMEM((C,), jnp.float32),
        pltpu.SemaphoreType.DMA,
        pltpu.SemaphoreType.DMA,
    ],
)
def _sc_shift(x_hbm, idx_hbm, out_hbm, buf, sav, idxv, edge, in_sem, out_sem):
    wid = lax.axis_index("s") * 2 + lax.axis_index("c")
    g = wid // NS_STRIP
    st = wid % NS_STRIP
    r0 = g * ROWS
    c0 = st * CHUNK
    lanes = lax.iota(jnp.int32, 16)

    pltpu.sync_copy(idx_hbm, idxv)

    def stream_in(r):
        slot = lax.rem(r, 4)
        pltpu.sync_copy(
            x_hbm.at[r, pl.ds(c0, CHUNK)], buf.at[pl.ds(slot * CHUNK, CHUNK)]
        )

    def issue_in(r):
        slot = lax.rem(r, 4)
        pltpu.async_copy(
            x_hbm.at[r, pl.ds(c0, CHUNK)], buf.at[pl.ds(slot * CHUNK, CHUNK)], in_sem
        )

    def issue_out(r):
        slot = lax.rem(r, 4)
        pltpu.async_copy(
            buf.at[pl.ds(slot * CHUNK, CHUNK)], out_hbm.at[r, pl.ds(c0, CHUNK)], out_sem
        )

    def wait_chunk(sem):
        pltpu.make_async_copy(
            x_hbm.at[0, pl.ds(0, CHUNK)], buf.at[pl.ds(0, CHUNK)], sem
        ).wait()

    def extract(r):
        slot = lax.rem(r, 4)

        def body(v, _):
            iv = idxv[pl.ds(v * 16, 16)]
            vals = plsc.load_gather(buf, [slot * CHUNK + iv])
            sav[pl.ds(slot * NSPEC + v * 16, 16)] = vals
            return 0

        lax.fori_loop(0, NV, body, 0)

    def fixup(r):
        slot = lax.rem(r, 4)
        sp1 = lax.rem(r + 1, 4)
        sm1 = lax.rem(r + 3, 4)

        def body(v, _):
            iv = idxv[pl.ds(v * 16, 16)]
            src_slot = jnp.where(v < NVH, sp1, sm1)
            vals = sav[pl.ds(src_slot * NSPEC + v * 16, 16)]
            plsc.store_scatter(buf, [slot * CHUNK + iv], vals)
            return 0

        lax.fori_loop(0, NV, body, 0)

    def stream_out(r):
        slot = lax.rem(r, 4)
        pltpu.sync_copy(
            buf.at[pl.ds(slot * CHUNK, CHUNK)], out_hbm.at[r, pl.ds(c0, CHUNK)]
        )

    def synth_m1():
        @pl.when(st > 0)
        def _():
            pltpu.sync_copy(x_hbm.at[0, pl.ds(c0 - C, C)], edge)

        def body(v, _):
            iv = lanes + (NSPEC // 2 + v * 16 - 1)
            vals = plsc.load_gather(sav, [iv])
            sav[pl.ds(3 * NSPEC + NSPEC // 2 + v * 16, 16)] = vals
            return 0

        lax.fori_loop(0, NVH, body, 0)
        bwd_ch = idxv[pl.ds(NSPEC, 16)]
        vals = plsc.load_gather(edge, [bwd_ch])
        pred = jnp.full((16,), st, jnp.int32) > 0
        vals = jnp.where(pred, vals, jnp.zeros_like(vals))
        plsc.store_scatter(
            sav, [3 * NSPEC + NSPEC // 2 + lanes * PX], vals, mask=lanes < 5
        )

    def synth_p1():
        @pl.when(st < NS_STRIP - 1)
        def _():
            pltpu.sync_copy(x_hbm.at[H - 1, pl.ds(c0 + CHUNK, C)], edge)

        def body(v, _):
            iv = lanes + (3 * NSPEC + v * 16 + 1)
            vals = plsc.load_gather(sav, [iv])
            sav[pl.ds(v * 16, 16)] = vals
            return 0

        lax.fori_loop(0, NVH, body, 0)
        fwd_ch = idxv[pl.ds(NSPEC + 16, 16)]
        vals = plsc.load_gather(edge, [fwd_ch])
        pred = jnp.full((16,), st, jnp.int32) < NS_STRIP - 1
        vals = jnp.where(pred, vals, jnp.zeros_like(vals))
        plsc.store_scatter(sav, [lanes * PX + (PX - 1)], vals, mask=lanes < 5)

    @pl.when(g > 0)
    def _():
        stream_in(r0 - 1)
        extract(r0 - 1)

    stream_in(r0)
    extract(r0)

    @pl.when(g == 0)
    def _():
        synth_m1()

    last_in = jnp.minimum(r0 + ROWS, H - 1)
    issue_in(r0 + 1)

    def row_body(i, _):
        r = r0 + i

        @pl.when(i >= 1)
        def _():
            wait_chunk(out_sem)

        @pl.when(r + 2 <= last_in)
        def _():
            issue_in(r + 2)

        @pl.when(r + 1 <= last_in)
        def _():
            wait_chunk(in_sem)
            extract(r + 1)

        @pl.when(r + 1 == H)
        def _():
            synth_p1()

        fixup(r)
        issue_out(r)
        return 0

    lax.fori_loop(0, ROWS, row_body, 0)
    wait_chunk(out_sem)


def kernel(x):
    x2d = x.reshape(H, WC)
    out2d = _sc_shift(x2d, jnp.asarray(_IDX))
    return out2d.reshape(H, W, C)


# speedup vs baseline: 9.4862x; 6.1516x over previous
import functools

import jax
import jax.numpy as jnp
import numpy as np
from jax import lax
from jax.experimental import pallas as pl
from jax.experimental.pallas import tpu as pltpu
from jax.experimental.pallas import tpu_sc as plsc

H, W, C = 512, 512, 224
N_SHIFT = 10

_rng = np.random.default_rng(0)
_allc = _rng.permutation(C)[:N_SHIFT]
_FWD = np.sort(_allc[: N_SHIFT // 2])
_BWD = np.sort(_allc[N_SHIFT // 2:])

NG = 8
NB = 4
ROWS = H // NG
CB = C // NB
CHUNK = CB * W

_FWD_LOC = [[c - CB * b for c in _FWD if c // CB == b] for b in range(NB)]
_BWD_LOC = [[c - CB * b for c in _BWD if c // CB == b] for b in range(NB)]
_NFWD = [len(f) for f in _FWD_LOC]
_NSPC = [len(f) + len(bw) for f, bw in zip(_FWD_LOC, _BWD_LOC)]
_MAXN = max(_NSPC)
SAVE_MAX = _MAXN * W

_CHTBL = np.zeros((NB, 16), np.int32)
for _b in range(NB):
    _row = _FWD_LOC[_b] + _BWD_LOC[_b]
    _CHTBL[_b, : len(_row)] = _row

_mesh = plsc.VectorSubcoreMesh(
    core_axis_name="c", subcore_axis_name="s", num_cores=2, num_subcores=16
)


def _sel(b, table):
    v = jnp.int32(table[NB - 1])
    for i in range(NB - 2, -1, -1):
        v = jnp.where(b == i, jnp.int32(table[i]), v)
    return v


@functools.partial(
    pl.kernel,
    out_type=jax.ShapeDtypeStruct((H, C, W), jnp.float32),
    mesh=_mesh,
    compiler_params=pltpu.CompilerParams(
        use_tc_tiling_on_sc=True, needs_layout_passes=False
    ),
    scratch_types=[
        pltpu.VMEM((4, CB, W), jnp.float32),
        pltpu.VMEM((4 * SAVE_MAX,), jnp.float32),
        pltpu.VMEM((16,), jnp.int32),
        pltpu.SemaphoreType.DMA,
        pltpu.SemaphoreType.DMA,
    ],
)
def _sc_shift(x_hbm, ch_hbm, out_hbm, buf, sav, chv, in_sem, out_sem):
    wid = lax.axis_index("s") * 2 + lax.axis_index("c")
    g = wid // NB
    b = wid % NB
    r0 = g * ROWS
    cb0 = b * CB
    nf = _sel(b, _NFWD)
    nb = _sel(b, _NSPC)
    lanes = lax.iota(jnp.int32, 16)

    pltpu.sync_copy(ch_hbm.at[b], chv)

    def splat(v):
        return jnp.full((16,), v, jnp.int32)

    def issue_in(r):
        slot = lax.rem(r, 4)
        pltpu.async_copy(x_hbm.at[r, pl.ds(cb0, CB), :], buf.at[slot], in_sem)

    def issue_out(r):
        slot = lax.rem(r, 4)
        pltpu.async_copy(buf.at[slot], out_hbm.at[r, pl.ds(cb0, CB), :], out_sem)

    def wait_chunk(sem):
        pltpu.make_async_copy(
            x_hbm.at[0, pl.ds(0, CB), :], buf.at[0], sem
        ).wait()

    def stream_in_sync(r):
        slot = lax.rem(r, 4)
        pltpu.sync_copy(x_hbm.at[r, pl.ds(cb0, CB), :], buf.at[slot])

    def extract(r):
        slot = lax.rem(r, 4)

        def outer(i, _):
            c_vec = plsc.load_gather(chv, [splat(i)])

            def inner(q, _):
                w_vec = q * 16 + lanes
                vals = plsc.load_gather(buf, [splat(slot), c_vec, w_vec])
                sav[pl.ds(slot * SAVE_MAX + i * W + q * 16, 16)] = vals
                return 0

            lax.fori_loop(0, W // 16, inner, 0)
            return 0

        lax.fori_loop(0, nb, outer, 0)

    def fixup(r):
        slot = lax.rem(r, 4)
        sp1 = lax.rem(r + 1, 4)
        sm1 = lax.rem(r + 3, 4)

        def outer(i, _):
            c_vec = plsc.load_gather(chv, [splat(i)])
            src = jnp.where(i < nf, sp1, sm1)

            def inner(q, _):
                w_vec = q * 16 + lanes
                vals = sav[pl.ds(src * SAVE_MAX + i * W + q * 16, 16)]
                plsc.store_scatter(buf, [splat(slot), c_vec, w_vec], vals)
                return 0

            lax.fori_loop(0, W // 16, inner, 0)
            return 0

        lax.fori_loop(0, nb, outer, 0)

    def synth_m1():
        def outer(i, _):
            def inner(q, _):
                iv = lanes + (i * W + q * 16 - 1)
                vals = plsc.load_gather(sav, [iv])
                sav[pl.ds(3 * SAVE_MAX + i * W + q * 16, 16)] = vals
                return 0

            lax.fori_loop(0, W // 16, inner, 0)
            return 0

        lax.fori_loop(nf, nb, outer, 0)
        plsc.store_scatter(
            sav,
            [3 * SAVE_MAX + (nf + lanes) * W],
            jnp.zeros((16,), jnp.float32),
            mask=lanes < nb - nf,
        )

    def synth_p1():
        def outer(i, _):
            def inner(q, _):
                iv = lanes + (3 * SAVE_MAX + i * W + q * 16 + 1)
                vals = plsc.load_gather(sav, [iv])
                sav[pl.ds(i * W + q * 16, 16)] = vals
                return 0

            lax.fori_loop(0, W // 16, inner, 0)
            return 0

        lax.fori_loop(0, nf, outer, 0)
        plsc.store_scatter(
            sav,
            [lanes * W + (W - 1)],
            jnp.zeros((16,), jnp.float32),
            mask=lanes < nf,
        )

    @pl.when(g > 0)
    def _():
        stream_in_sync(r0 - 1)
        extract(r0 - 1)

    stream_in_sync(r0)
    extract(r0)

    @pl.when(g == 0)
    def _():
        synth_m1()

    last_in = jnp.minimum(r0 + ROWS, H - 1)
    issue_in(r0 + 1)

    def row_body(i, _):
        r = r0 + i

        @pl.when(i >= 1)
        def _():
            wait_chunk(out_sem)

        @pl.when(r + 2 <= last_in)
        def _():
            issue_in(r + 2)

        @pl.when(r + 1 <= last_in)
        def _():
            wait_chunk(in_sem)
            extract(r + 1)

        @pl.when(r + 1 == H)
        def _():
            synth_p1()

        fixup(r)
        issue_out(r)
        return 0

    lax.fori_loop(0, ROWS, row_body, 0)
    wait_chunk(out_sem)


def kernel(x):
    xt = jnp.transpose(x, (0, 2, 1))
    ot = _sc_shift(xt, jnp.asarray(_CHTBL))
    return jnp.transpose(ot, (0, 2, 1))


# speedup vs baseline: 9.8992x; 1.0435x over previous
import functools

import jax
import jax.numpy as jnp
import numpy as np
from jax import lax
from jax.experimental import pallas as pl
from jax.experimental.pallas import tpu as pltpu
from jax.experimental.pallas import tpu_sc as plsc

H, W, C = 512, 512, 224
N_SHIFT = 10

_rng = np.random.default_rng(0)
_allc = _rng.permutation(C)[:N_SHIFT]
_FWD = np.sort(_allc[: N_SHIFT // 2])
_BWD = np.sort(_allc[N_SHIFT // 2:])

NG = 8
NB = 4
ROWS = H // NG
CB = C // NB
CHUNK = CB * W

_FWD_LOC = [[c - CB * b for c in _FWD if c // CB == b] for b in range(NB)]
_BWD_LOC = [[c - CB * b for c in _BWD if c // CB == b] for b in range(NB)]
_NFWD = [len(f) for f in _FWD_LOC]
_NSPC = [len(f) + len(bw) for f, bw in zip(_FWD_LOC, _BWD_LOC)]
_MAXN = max(_NSPC)
SAVE_MAX = _MAXN * W

_CHTBL = np.zeros((NB, 16), np.int32)
for _b in range(NB):
    _row = _FWD_LOC[_b] + _BWD_LOC[_b]
    _CHTBL[_b, : len(_row)] = _row

_mesh = plsc.VectorSubcoreMesh(
    core_axis_name="c", subcore_axis_name="s", num_cores=2, num_subcores=16
)


def _sel(b, table):
    v = jnp.int32(table[NB - 1])
    for i in range(NB - 2, -1, -1):
        v = jnp.where(b == i, jnp.int32(table[i]), v)
    return v


@functools.partial(
    pl.kernel,
    out_type=jax.ShapeDtypeStruct((H, C, W), jnp.float32),
    mesh=_mesh,
    compiler_params=pltpu.CompilerParams(
        use_tc_tiling_on_sc=True, needs_layout_passes=False
    ),
    scratch_types=[
        pltpu.VMEM((4, CB, W), jnp.float32),
        pltpu.VMEM((4 * SAVE_MAX,), jnp.float32),
        pltpu.VMEM((16,), jnp.int32),
        pltpu.SemaphoreType.DMA,
        pltpu.SemaphoreType.DMA,
    ],
)
def _sc_shift(x_hbm, ch_hbm, out_hbm, buf, sav, chv, in_sem, out_sem):
    wid = lax.axis_index("s") * 2 + lax.axis_index("c")
    g = wid % NG
    b = wid // NG
    r0 = g * ROWS
    cb0 = b * CB
    nf = _sel(b, _NFWD)
    nb = _sel(b, _NSPC)
    lanes = lax.iota(jnp.int32, 16)

    pltpu.sync_copy(ch_hbm.at[b], chv)

    def splat(v):
        return jnp.full((16,), v, jnp.int32)

    def issue_in(r):
        slot = lax.rem(r, 4)
        pltpu.async_copy(x_hbm.at[r, pl.ds(cb0, CB), :], buf.at[slot], in_sem)

    def issue_out(r):
        slot = lax.rem(r, 4)
        pltpu.async_copy(buf.at[slot], out_hbm.at[r, pl.ds(cb0, CB), :], out_sem)

    def wait_chunk(sem):
        pltpu.make_async_copy(
            x_hbm.at[0, pl.ds(0, CB), :], buf.at[0], sem
        ).wait()

    def stream_in_sync(r):
        slot = lax.rem(r, 4)
        pltpu.sync_copy(x_hbm.at[r, pl.ds(cb0, CB), :], buf.at[slot])

    def extract(r):
        slot = lax.rem(r, 4)

        def outer(i, _):
            c_vec = plsc.load_gather(chv, [splat(i)])

            def inner(q, _):
                w_vec = q * 16 + lanes
                vals = plsc.load_gather(buf, [splat(slot), c_vec, w_vec])
                sav[pl.ds(slot * SAVE_MAX + i * W + q * 16, 16)] = vals
                return 0

            lax.fori_loop(0, W // 16, inner, 0)
            return 0

        lax.fori_loop(0, nb, outer, 0)

    def fixup(r):
        slot = lax.rem(r, 4)
        sp1 = lax.rem(r + 1, 4)
        sm1 = lax.rem(r + 3, 4)

        def outer(i, _):
            c_vec = plsc.load_gather(chv, [splat(i)])
            src = jnp.where(i < nf, sp1, sm1)

            def inner(q, _):
                w_vec = q * 16 + lanes
                vals = sav[pl.ds(src * SAVE_MAX + i * W + q * 16, 16)]
                plsc.store_scatter(buf, [splat(slot), c_vec, w_vec], vals)
                return 0

            lax.fori_loop(0, W // 16, inner, 0)
            return 0

        lax.fori_loop(0, nb, outer, 0)

    def synth_m1():
        def outer(i, _):
            def inner(q, _):
                iv = lanes + (i * W + q * 16 - 1)
                vals = plsc.load_gather(sav, [iv])
                sav[pl.ds(3 * SAVE_MAX + i * W + q * 16, 16)] = vals
                return 0

            lax.fori_loop(0, W // 16, inner, 0)
            return 0

        lax.fori_loop(nf, nb, outer, 0)
        plsc.store_scatter(
            sav,
            [3 * SAVE_MAX + (nf + lanes) * W],
            jnp.zeros((16,), jnp.float32),
            mask=lanes < nb - nf,
        )

    def synth_p1():
        def outer(i, _):
            def inner(q, _):
                iv = lanes + (3 * SAVE_MAX + i * W + q * 16 + 1)
                vals = plsc.load_gather(sav, [iv])
                sav[pl.ds(i * W + q * 16, 16)] = vals
                return 0

            lax.fori_loop(0, W // 16, inner, 0)
            return 0

        lax.fori_loop(0, nf, outer, 0)
        plsc.store_scatter(
            sav,
            [lanes * W + (W - 1)],
            jnp.zeros((16,), jnp.float32),
            mask=lanes < nf,
        )

    @pl.when(g > 0)
    def _():
        stream_in_sync(r0 - 1)
        extract(r0 - 1)

    stream_in_sync(r0)
    extract(r0)

    @pl.when(g == 0)
    def _():
        synth_m1()

    last_in = jnp.minimum(r0 + ROWS, H - 1)
    issue_in(r0 + 1)

    def row_body(i, _):
        r = r0 + i

        @pl.when(i >= 1)
        def _():
            wait_chunk(out_sem)

        @pl.when(r + 2 <= last_in)
        def _():
            issue_in(r + 2)

        @pl.when(r + 1 <= last_in)
        def _():
            wait_chunk(in_sem)
            extract(r + 1)

        @pl.when(r + 1 == H)
        def _():
            synth_p1()

        fixup(r)
        issue_out(r)
        return 0

    lax.fori_loop(0, ROWS, row_body, 0)
    wait_chunk(out_sem)


def kernel(x):
    xt = jnp.transpose(x, (0, 2, 1))
    ot = _sc_shift(xt, jnp.asarray(_CHTBL))
    return jnp.transpose(ot, (0, 2, 1))


# speedup vs baseline: 9.9415x; 1.0043x over previous
import functools

import jax
import jax.numpy as jnp
import numpy as np
from jax import lax
from jax.experimental import pallas as pl
from jax.experimental.pallas import tpu as pltpu
from jax.experimental.pallas import tpu_sc as plsc

H, W, C = 512, 512, 224
N_SHIFT = 10

_rng = np.random.default_rng(0)
_allc = _rng.permutation(C)[:N_SHIFT]
_FWD = np.sort(_allc[: N_SHIFT // 2])
_BWD = np.sort(_allc[N_SHIFT // 2:])

NG = 8
NB = 4
ROWS = H // NG
CB = C // NB
CHUNK = CB * W

_FWD_LOC = [[c - CB * b for c in _FWD if c // CB == b] for b in range(NB)]
_BWD_LOC = [[c - CB * b for c in _BWD if c // CB == b] for b in range(NB)]
_NFWD = [len(f) for f in _FWD_LOC]
_NSPC = [len(f) + len(bw) for f, bw in zip(_FWD_LOC, _BWD_LOC)]
_MAXN = max(_NSPC)
SAVE_MAX = _MAXN * W

_CHTBL = np.zeros((NB, 16), np.int32)
for _b in range(NB):
    _row = _FWD_LOC[_b] + _BWD_LOC[_b]
    _CHTBL[_b, : len(_row)] = _row

_mesh = plsc.VectorSubcoreMesh(
    core_axis_name="c", subcore_axis_name="s", num_cores=2, num_subcores=16
)


def _sel(b, table):
    v = jnp.int32(table[NB - 1])
    for i in range(NB - 2, -1, -1):
        v = jnp.where(b == i, jnp.int32(table[i]), v)
    return v


@functools.partial(
    pl.kernel,
    out_type=jax.ShapeDtypeStruct((H, C, W), jnp.float32),
    mesh=_mesh,
    compiler_params=pltpu.CompilerParams(
        use_tc_tiling_on_sc=True, needs_layout_passes=False
    ),
    scratch_types=[
        pltpu.VMEM((4, CB, W), jnp.float32),
        pltpu.VMEM((4 * SAVE_MAX,), jnp.float32),
        pltpu.SemaphoreType.DMA,
        pltpu.SemaphoreType.DMA,
    ],
)
def _sc_shift(x_hbm, out_hbm, buf, sav, in_sem, out_sem):
    wid = lax.axis_index("s") * 2 + lax.axis_index("c")
    g = wid % NG
    b = wid // NG
    r0 = g * ROWS
    cb0 = b * CB
    nf = _sel(b, _NFWD)
    nb = _sel(b, _NSPC)
    lanes = lax.iota(jnp.int32, 16)

    def issue_in(r):
        slot = lax.rem(r, 4)
        pltpu.async_copy(x_hbm.at[r, pl.ds(cb0, CB), :], buf.at[slot], in_sem)

    def issue_out(r):
        slot = lax.rem(r, 4)
        pltpu.async_copy(buf.at[slot], out_hbm.at[r, pl.ds(cb0, CB), :], out_sem)

    def wait_chunk(sem):
        pltpu.make_async_copy(
            x_hbm.at[0, pl.ds(0, CB), :], buf.at[0], sem
        ).wait()

    def stream_in_sync(r):
        slot = lax.rem(r, 4)
        pltpu.sync_copy(x_hbm.at[r, pl.ds(cb0, CB), :], buf.at[slot])

    def extract(r):
        slot = lax.rem(r, 4)
        for i in range(_MAXN):
            @pl.when(i < nb)
            def _(i=i):
                c = _sel(b, [int(_CHTBL[bb][i]) for bb in range(NB)])

                def inner(q, _):
                    vals = buf[slot, c, pl.ds(q * 16, 16)]
                    sav[pl.ds(slot * SAVE_MAX + i * W + q * 16, 16)] = vals
                    return 0

                lax.fori_loop(0, W // 16, inner, 0, unroll=8)

    def fixup(r):
        slot = lax.rem(r, 4)
        sp1 = lax.rem(r + 1, 4)
        sm1 = lax.rem(r + 3, 4)
        for i in range(_MAXN):
            @pl.when(i < nb)
            def _(i=i):
                c = _sel(b, [int(_CHTBL[bb][i]) for bb in range(NB)])
                src = jnp.where(i < nf, sp1, sm1)

                def inner(q, _):
                    vals = sav[pl.ds(src * SAVE_MAX + i * W + q * 16, 16)]
                    buf[slot, c, pl.ds(q * 16, 16)] = vals
                    return 0

                lax.fori_loop(0, W // 16, inner, 0, unroll=8)

    def synth_m1():
        def outer(i, _):
            def inner(q, _):
                iv = lanes + (i * W + q * 16 - 1)
                vals = plsc.load_gather(sav, [iv])
                sav[pl.ds(3 * SAVE_MAX + i * W + q * 16, 16)] = vals
                return 0

            lax.fori_loop(0, W // 16, inner, 0)
            return 0

        lax.fori_loop(nf, nb, outer, 0)
        plsc.store_scatter(
            sav,
            [3 * SAVE_MAX + (nf + lanes) * W],
            jnp.zeros((16,), jnp.float32),
            mask=lanes < nb - nf,
        )

    def synth_p1():
        def outer(i, _):
            def inner(q, _):
                iv = lanes + (3 * SAVE_MAX + i * W + q * 16 + 1)
                vals = plsc.load_gather(sav, [iv])
                sav[pl.ds(i * W + q * 16, 16)] = vals
                return 0

            lax.fori_loop(0, W // 16, inner, 0)
            return 0

        lax.fori_loop(0, nf, outer, 0)
        plsc.store_scatter(
            sav,
            [lanes * W + (W - 1)],
            jnp.zeros((16,), jnp.float32),
            mask=lanes < nf,
        )

    @pl.when(g > 0)
    def _():
        stream_in_sync(r0 - 1)
        extract(r0 - 1)

    stream_in_sync(r0)
    extract(r0)

    @pl.when(g == 0)
    def _():
        synth_m1()

    last_in = jnp.minimum(r0 + ROWS, H - 1)
    issue_in(r0 + 1)

    def row_body(i, _):
        r = r0 + i

        @pl.when(i >= 1)
        def _():
            wait_chunk(out_sem)

        @pl.when(r + 2 <= last_in)
        def _():
            issue_in(r + 2)

        @pl.when(r + 1 <= last_in)
        def _():
            wait_chunk(in_sem)
            extract(r + 1)

        @pl.when(r + 1 == H)
        def _():
            synth_p1()

        fixup(r)
        issue_out(r)
        return 0

    lax.fori_loop(0, ROWS, row_body, 0)
    wait_chunk(out_sem)


def kernel(x):
    xt = jnp.transpose(x, (0, 2, 1))
    ot = _sc_shift(xt)
    return jnp.transpose(ot, (0, 2, 1))


# speedup vs baseline: 10.0765x; 1.0136x over previous
import functools

import jax
import jax.numpy as jnp
import numpy as np
from jax import lax
from jax.experimental import pallas as pl
from jax.experimental.pallas import tpu as pltpu
from jax.experimental.pallas import tpu_sc as plsc

H, W, C = 512, 512, 224
N_SHIFT = 10

_rng = np.random.default_rng(0)
_allc = _rng.permutation(C)[:N_SHIFT]
_FWD = np.sort(_allc[: N_SHIFT // 2])
_BWD = np.sort(_allc[N_SHIFT // 2:])

NG = 8
NB = 4
ROWS = H // NG
CB = C // NB
CHUNK = CB * W

_FWD_LOC = [[c - CB * b for c in _FWD if c // CB == b] for b in range(NB)]
_BWD_LOC = [[c - CB * b for c in _BWD if c // CB == b] for b in range(NB)]
_NFWD = [len(f) for f in _FWD_LOC]
_NSPC = [len(f) + len(bw) for f, bw in zip(_FWD_LOC, _BWD_LOC)]
_MAXN = max(_NSPC)
SAVE_MAX = _MAXN * W

_CHTBL = np.zeros((NB, 16), np.int32)
for _b in range(NB):
    _row = _FWD_LOC[_b] + _BWD_LOC[_b]
    _CHTBL[_b, : len(_row)] = _row

_mesh = plsc.VectorSubcoreMesh(
    core_axis_name="c", subcore_axis_name="s", num_cores=2, num_subcores=16
)


def _sel(b, table):
    v = jnp.int32(table[NB - 1])
    for i in range(NB - 2, -1, -1):
        v = jnp.where(b == i, jnp.int32(table[i]), v)
    return v


@functools.partial(
    pl.kernel,
    out_type=jax.ShapeDtypeStruct((H, C, W), jnp.float32),
    mesh=_mesh,
    compiler_params=pltpu.CompilerParams(
        use_tc_tiling_on_sc=True, needs_layout_passes=False
    ),
    scratch_types=[
        pltpu.VMEM((4, CB, W), jnp.float32),
        pltpu.VMEM((4 * SAVE_MAX,), jnp.float32),
        pltpu.SemaphoreType.DMA,
        pltpu.SemaphoreType.DMA,
    ],
)
def _sc_shift(x_hbm, out_hbm, buf, sav, in_sem, out_sem):
    wid = lax.axis_index("s") * 2 + lax.axis_index("c")
    g = wid % NG
    b = wid // NG
    r0 = g * ROWS
    cb0 = b * CB
    nf = _sel(b, _NFWD)
    nb = _sel(b, _NSPC)
    lanes = lax.iota(jnp.int32, 16)

    def issue_in(r):
        slot = lax.rem(r, 4)
        pltpu.async_copy(x_hbm.at[r, pl.ds(cb0, CB), :], buf.at[slot], in_sem)

    def issue_out(r):
        slot = lax.rem(r, 4)
        pltpu.async_copy(buf.at[slot], out_hbm.at[r, pl.ds(cb0, CB), :], out_sem)

    def wait_chunk(sem):
        pltpu.make_async_copy(
            x_hbm.at[0, pl.ds(0, CB), :], buf.at[0], sem
        ).wait()

    def extract(r):
        slot = lax.rem(r, 4)
        for i in range(_MAXN):
            @pl.when(i < nb)
            def _(i=i):
                c = _sel(b, [int(_CHTBL[bb][i]) for bb in range(NB)])

                def inner(q, _):
                    vals = buf[slot, c, pl.ds(q * 16, 16)]
                    sav[pl.ds(slot * SAVE_MAX + i * W + q * 16, 16)] = vals
                    return 0

                lax.fori_loop(0, W // 16, inner, 0, unroll=8)

    def fixup_half(r, fwd):
        slot = lax.rem(r, 4)
        src = lax.rem(r + 1, 4) if fwd else lax.rem(r + 3, 4)
        for i in range(_MAXN):
            half_ok = (i < nf) if fwd else jnp.logical_and(i >= nf, i < nb)

            @pl.when(half_ok)
            def _(i=i):
                c = _sel(b, [int(_CHTBL[bb][i]) for bb in range(NB)])

                def inner(q, _):
                    vals = sav[pl.ds(src * SAVE_MAX + i * W + q * 16, 16)]
                    buf[slot, c, pl.ds(q * 16, 16)] = vals
                    return 0

                lax.fori_loop(0, W // 16, inner, 0, unroll=8)

    def synth_m1():
        def outer(i, _):
            def inner(q, _):
                iv = lanes + (i * W + q * 16 - 1)
                vals = plsc.load_gather(sav, [iv])
                sav[pl.ds(3 * SAVE_MAX + i * W + q * 16, 16)] = vals
                return 0

            lax.fori_loop(0, W // 16, inner, 0)
            return 0

        lax.fori_loop(nf, nb, outer, 0)
        plsc.store_scatter(
            sav,
            [3 * SAVE_MAX + (nf + lanes) * W],
            jnp.zeros((16,), jnp.float32),
            mask=lanes < nb - nf,
        )

    def synth_p1():
        def outer(i, _):
            def inner(q, _):
                iv = lanes + (3 * SAVE_MAX + i * W + q * 16 + 1)
                vals = plsc.load_gather(sav, [iv])
                sav[pl.ds(i * W + q * 16, 16)] = vals
                return 0

            lax.fori_loop(0, W // 16, inner, 0)
            return 0

        lax.fori_loop(0, nf, outer, 0)
        plsc.store_scatter(
            sav,
            [lanes * W + (W - 1)],
            jnp.zeros((16,), jnp.float32),
            mask=lanes < nf,
        )

    @pl.when(g > 0)
    def _():
        issue_in(r0 - 1)

    issue_in(r0)
    issue_in(r0 + 1)

    @pl.when(g > 0)
    def _():
        wait_chunk(in_sem)
        extract(r0 - 1)

    wait_chunk(in_sem)
    extract(r0)

    @pl.when(g == 0)
    def _():
        synth_m1()

    last_in = jnp.minimum(r0 + ROWS, H - 1)

    def row_body(i, _):
        r = r0 + i

        @pl.when(i >= 1)
        def _():
            wait_chunk(out_sem)

        @pl.when(r + 2 <= last_in)
        def _():
            issue_in(r + 2)

        fixup_half(r, fwd=False)

        @pl.when(r + 1 <= last_in)
        def _():
            wait_chunk(in_sem)
            extract(r + 1)

        @pl.when(r + 1 == H)
        def _():
            synth_p1()

        fixup_half(r, fwd=True)
        issue_out(r)
        return 0

    lax.fori_loop(0, ROWS, row_body, 0)
    wait_chunk(out_sem)


def kernel(x):
    xt = jnp.transpose(x, (0, 2, 1))
    ot = _sc_shift(xt)
    return jnp.transpose(ot, (0, 2, 1))
